# baseline (speedup 1.0000x reference)
import functools

import jax
import jax.numpy as jnp
import numpy as np
from jax import lax
from jax.experimental import pallas as pl
from jax.experimental.pallas import tpu as pltpu
from jax.experimental.pallas import tpu_sc as plsc

_NUM_EMB = 1000000
_DIM = 32
_QMAX = 31.0
_B = 16384 * 50
_NW = 32
_ROWS_PER_W = _B // _NW
_CHUNK = 1600
_NCHUNKS = _ROWS_PER_W // _CHUNK
_MAGIC = 1.5 * 2.0 ** 23


def _round_ne(x):
    return (x + _MAGIC) - _MAGIC


_GATHER_DNUMS = lax.GatherDimensionNumbers(
    offset_dims=(), collapsed_slice_dims=(0,), start_index_map=(0,)
)


def _shuffle(v, idx):
    return lax.gather(
        v,
        idx[:, None],
        _GATHER_DNUMS,
        slice_sizes=(1,),
        mode=lax.GatherScatterMode.PROMISE_IN_BOUNDS,
    )


def _lane_max_all(m):
    lanes = lax.iota(jnp.int32, 16)
    for d in (8, 4, 2, 1):
        m = jnp.maximum(m, _shuffle(m, lanes ^ d))
    return m


def _quantize_rows(buf, n_rows):
    def body(r, _):
        v0 = buf[r, pl.ds(0, 16)]
        v1 = buf[r, pl.ds(16, 16)]
        m = _lane_max_all(jnp.maximum(jnp.abs(v0), jnp.abs(v1)))
        scale = m / _QMAX
        scale = jnp.where(scale == 0.0, 1.0, scale)
        inv = 1.0 / scale
        buf[r, pl.ds(0, 16)] = _round_ne(v0 * inv) * scale
        buf[r, pl.ds(16, 16)] = _round_ne(v1 * inv) * scale
        return _

    lax.fori_loop(0, n_rows, body, None)


def _make_kernel():
    mesh = plsc.VectorSubcoreMesh(core_axis_name="c", subcore_axis_name="s")

    @functools.partial(
        pl.kernel,
        out_type=jax.ShapeDtypeStruct((_B, _DIM), jnp.float32),
        mesh=mesh,
        scratch_types=[
            pltpu.VMEM((_CHUNK,), jnp.int32),
            pltpu.VMEM((_CHUNK, _DIM), jnp.float32),
            pltpu.SemaphoreType.DMA,
        ],
        compiler_params=pltpu.CompilerParams(use_tc_tiling_on_sc=False),
    )
    def k(table_hbm, idx_hbm, out_hbm, idx_v, buf, sem):
        wid = lax.axis_index("s") * 2 + lax.axis_index("c")
        w_base = wid * _ROWS_PER_W

        def chunk_body(t, _):
            base = w_base + t * _CHUNK
            pltpu.sync_copy(idx_hbm.at[pl.ds(base, _CHUNK)], idx_v)
            pltpu.async_copy(table_hbm.at[idx_v], buf, sem).wait()
            _quantize_rows(buf, _CHUNK)
            pltpu.sync_copy(buf, out_hbm.at[pl.ds(base, _CHUNK)])
            return _

        lax.fori_loop(0, _NCHUNKS, chunk_body, None)

    return k


_kernel_call = _make_kernel()


@jax.jit
def kernel(input, weight):
    idx = input.reshape(-1).astype(jnp.int32)
    out = _kernel_call(weight, idx)
    return out.reshape(input.shape + (_DIM,))


# speedup vs baseline: 1.3225x; 1.3225x over previous
import functools

import jax
import jax.numpy as jnp
from jax import lax
from jax.experimental import pallas as pl
from jax.experimental.pallas import tpu as pltpu
from jax.experimental.pallas import tpu_sc as plsc

_DIM = 32
_QMAX = 31.0
_B = 16384 * 50
_NW = 32
_ROWS_PER_W = _B // _NW
_CHUNK = 1600
_NCHUNKS = _ROWS_PER_W // _CHUNK
_NBUF = 2
_MAGIC = 1.5 * 2.0 ** 23


def _round_ne(x):
    return (x + _MAGIC) - _MAGIC


_GATHER_DNUMS = lax.GatherDimensionNumbers(
    offset_dims=(), collapsed_slice_dims=(0,), start_index_map=(0,)
)


def _shuffle(v, idx):
    return lax.gather(
        v,
        idx[:, None],
        _GATHER_DNUMS,
        slice_sizes=(1,),
        mode=lax.GatherScatterMode.PROMISE_IN_BOUNDS,
    )


def _lane_max_all(m):
    lanes = lax.iota(jnp.int32, 16)
    for d in (8, 4, 2, 1):
        m = jnp.maximum(m, _shuffle(m, lanes ^ d))
    return m


def _quantize_rows(buf, n_rows):
    @plsc.parallel_loop(0, n_rows, unroll=4)
    def body(r):
        v0 = buf[r, pl.ds(0, 16)]
        v1 = buf[r, pl.ds(16, 16)]
        m = _lane_max_all(jnp.maximum(jnp.abs(v0), jnp.abs(v1)))
        scale = m / _QMAX
        scale = jnp.where(scale == 0.0, 1.0, scale)
        inv = 1.0 / scale
        buf[r, pl.ds(0, 16)] = _round_ne(v0 * inv) * scale
        buf[r, pl.ds(16, 16)] = _round_ne(v1 * inv) * scale


def _make_kernel():
    mesh = plsc.VectorSubcoreMesh(core_axis_name="c", subcore_axis_name="s")

    @functools.partial(
        pl.kernel,
        out_type=jax.ShapeDtypeStruct((_B, _DIM), jnp.float32),
        mesh=mesh,
        scratch_types=[
            pltpu.VMEM((_CHUNK,), jnp.int32),
            pltpu.VMEM((_CHUNK,), jnp.int32),
            pltpu.VMEM((_CHUNK, _DIM), jnp.float32),
            pltpu.VMEM((_CHUNK, _DIM), jnp.float32),
            pltpu.SemaphoreType.DMA,
            pltpu.SemaphoreType.DMA,
            pltpu.SemaphoreType.DMA,
            pltpu.SemaphoreType.DMA,
        ],
        compiler_params=pltpu.CompilerParams(use_tc_tiling_on_sc=False),
    )
    def k(table_hbm, idx_hbm, out_hbm, idx0, idx1, buf0, buf1,
          gsem0, gsem1, wsem0, wsem1):
        wid = lax.axis_index("s") * 2 + lax.axis_index("c")
        w_base = wid * _ROWS_PER_W
        idx_v = (idx0, idx1)
        buf = (buf0, buf1)
        gsem = (gsem0, gsem1)
        wsem = (wsem0, wsem1)

        gathers = {}
        writes = {}
        for t in range(_NCHUNKS + 1):
            b = t % _NBUF
            if t < _NCHUNKS:
                if t >= _NBUF:
                    writes.pop(t - _NBUF).wait()
                base = w_base + t * _CHUNK
                pltpu.sync_copy(idx_hbm.at[pl.ds(base, _CHUNK)], idx_v[b])
                gathers[t] = pltpu.async_copy(
                    table_hbm.at[idx_v[b]], buf[b], gsem[b]
                )
            if t >= 1:
                tc = t - 1
                bc = tc % _NBUF
                gathers.pop(tc).wait()
                _quantize_rows(buf[bc], _CHUNK)
                writes[tc] = pltpu.async_copy(
                    buf[bc],
                    out_hbm.at[pl.ds(w_base + tc * _CHUNK, _CHUNK)],
                    wsem[bc],
                )
        for w in writes.values():
            w.wait()

    return k


_kernel_call = _make_kernel()


@jax.jit
def kernel(input, weight):
    idx = input.reshape(-1).astype(jnp.int32)
    out = _kernel_call(weight, idx)
    return out.reshape(input.shape + (_DIM,))
